# Initial kernel scaffold; baseline (speedup 1.0000x reference)
import jax
import jax.numpy as jnp
from jax.experimental import pallas as pl


def kernel(x_user, x_item, edge_index, norm_ui, norm_iu, W1_w, W1_b, W2_w, W2_b):
    raise NotImplementedError("write your pallas kernel here")


# speedup vs baseline: 4.5231x; 4.5231x over previous
import functools

import jax
import jax.numpy as jnp
from jax import lax
from jax.experimental import pallas as pl
from jax.experimental.pallas import tpu as pltpu
from jax.experimental.pallas import tpu_sc as plsc

N = 5000
D = 128
DP = 144
E = 320000
NC, NS, LANES = 2, 16, 16
NW = NC * NS
BLK = 128
NB = -(-E // (NW * BLK))
EPAD = NW * NB * BLK
NPAD = 5120
STRIPE = NPAD // NS


def _sc_segment_sums(xu_p, xi_p, idx, nrm, zeros):
  mesh = plsc.VectorSubcoreMesh(core_axis_name="c", subcore_axis_name="s")

  @functools.partial(
      pl.kernel,
      out_type=(
          jax.ShapeDtypeStruct((NC, NPAD, DP), jnp.float32),
          jax.ShapeDtypeStruct((NC, NPAD, DP), jnp.float32),
      ),
      mesh=mesh,
      compiler_params=pltpu.CompilerParams(use_tc_tiling_on_sc=False),
      scratch_types=[
          pltpu.VMEM((2, BLK), jnp.int32),
          pltpu.VMEM((2, BLK), jnp.float32),
          pltpu.VMEM((BLK, DP), jnp.float32),
          pltpu.VMEM((BLK, DP), jnp.float32),
          pltpu.VMEM_SHARED((NPAD, DP), jnp.float32),
          pltpu.VMEM_SHARED((NPAD, DP), jnp.float32),
          pltpu.SemaphoreType.DMA,
          pltpu.SemaphoreType.DMA,
      ],
  )
  def k(xu_hbm, xi_hbm, idx_hbm, nrm_hbm, z_hbm,
        accu_out, acci_out,
        edge_v, nrm_v, rows_u, rows_i, acc_u, acc_i,
        sem_u, sem_i):
    cid = lax.axis_index("c")
    sid = lax.axis_index("s")
    wid = cid * NS + sid
    stripe = pl.ds(sid * STRIPE, STRIPE)
    pltpu.sync_copy(z_hbm, acc_u.at[stripe])
    pltpu.sync_copy(z_hbm, acc_i.at[stripe])
    plsc.subcore_barrier()

    def body(b, carry):
      pltpu.sync_copy(idx_hbm.at[wid, b], edge_v)
      pltpu.sync_copy(nrm_hbm.at[wid, b], nrm_v)
      cp_u = pltpu.async_copy(xu_hbm.at[edge_v.at[0]], rows_u, sem_u)
      cp_i = pltpu.async_copy(xi_hbm.at[edge_v.at[1]], rows_i, sem_i)
      cp_u.wait()
      cp_i.wait()

      def scale(g, c2):
        sl16 = pl.ds(g * LANES, LANES)
        n16u = nrm_v[0, sl16]
        n16i = nrm_v[1, sl16]
        for j in range(LANES):
          r = g * LANES + j
          jv = jnp.full((LANES, 1), j, jnp.int32)
          dn = lax.GatherDimensionNumbers(
              offset_dims=(), collapsed_slice_dims=(0,), start_index_map=(0,))
          nu = lax.gather(n16u, jv, dn, (1,),
                          mode=lax.GatherScatterMode.PROMISE_IN_BOUNDS)
          ni = lax.gather(n16i, jv, dn, (1,),
                          mode=lax.GatherScatterMode.PROMISE_IN_BOUNDS)
          for kk in range(DP // LANES):
            sl = pl.ds(kk * LANES, LANES)
            rows_u[r, sl] = rows_u[r, sl] * nu
            rows_i[r, sl] = rows_i[r, sl] * ni
        return c2

      lax.fori_loop(0, BLK // LANES, scale, 0)
      pltpu.sync_copy(rows_u, acc_i.at[edge_v.at[1]], add=True)
      pltpu.sync_copy(rows_i, acc_u.at[edge_v.at[0]], add=True)
      return carry

    lax.fori_loop(0, NB, body, 0)
    plsc.subcore_barrier()
    pltpu.sync_copy(acc_u.at[stripe], accu_out.at[cid, stripe])
    pltpu.sync_copy(acc_i.at[stripe], acci_out.at[cid, stripe])

  return k(xu_p, xi_p, idx, nrm, zeros)


def _tc_finish(acc_u, acc_i, x_user, x_item, W1_w, W2_w, bsum):
  BR = 512
  grid = (-(-N // BR),)

  def body(au0, au1, ai0, ai1, xu, xi, w1, w2, bs, hu, hi):
    def one(a0, a1, x, out):
      a = a0[...] + a1[...]
      s = a[:, :D]
      c = a[:, D:D + 1]
      h = lax.dot_general(s, w1[...], (((1,), (1,)), ((), ())),
                          preferred_element_type=jnp.float32)
      h = h + lax.dot_general(x[...] * s, w2[...], (((1,), (1,)), ((), ())),
                              preferred_element_type=jnp.float32)
      h = h + c * bs[...]
      h = jnp.where(h >= 0, h, 0.2 * h)
      nrm = jnp.sqrt(jnp.sum(h * h, axis=1, keepdims=True))
      out[...] = h / jnp.maximum(nrm, 1e-12)

    one(au0, au1, xu, hu)
    one(ai0, ai1, xi, hi)

  bs_a = pl.BlockSpec((BR, DP), lambda i: (i, 0))
  bs_x = pl.BlockSpec((BR, D), lambda i: (i, 0))
  bs_w = pl.BlockSpec((D, D), lambda i: (0, 0))
  bs_b = pl.BlockSpec((1, D), lambda i: (0, 0))
  return pl.pallas_call(
      body,
      grid=grid,
      in_specs=[bs_a, bs_a, bs_a, bs_a, bs_x, bs_x, bs_w, bs_w, bs_b],
      out_specs=[bs_x, bs_x],
      out_shape=(jax.ShapeDtypeStruct((N, D), jnp.float32),
                 jax.ShapeDtypeStruct((N, D), jnp.float32)),
  )(acc_u[0], acc_u[1], acc_i[0], acc_i[1], x_user, x_item, W1_w, W2_w, bsum)


def kernel(x_user, x_item, edge_index, norm_ui, norm_iu, W1_w, W1_b, W2_w, W2_b):
  src = edge_index[0].astype(jnp.int32)
  dst = edge_index[1].astype(jnp.int32)
  pad = EPAD - E
  zi = jnp.zeros((pad,), jnp.int32)
  zf = jnp.zeros((pad,), jnp.float32)
  src3 = jnp.concatenate([src, zi]).reshape(NW, NB, 1, BLK)
  dst3 = jnp.concatenate([dst, zi]).reshape(NW, NB, 1, BLK)
  nui3 = jnp.concatenate([norm_ui[:, 0], zf]).reshape(NW, NB, 1, BLK)
  niu3 = jnp.concatenate([norm_iu[:, 0], zf]).reshape(NW, NB, 1, BLK)
  idx = jnp.concatenate([src3, dst3], axis=2)
  nrm = jnp.concatenate([nui3, niu3], axis=2)
  ones = jnp.ones((N, 1), jnp.float32)
  zpad = jnp.zeros((N, DP - D - 1), jnp.float32)
  xu_p = jnp.concatenate([x_user, ones, zpad], axis=1)
  xi_p = jnp.concatenate([x_item, ones, zpad], axis=1)
  zeros = jnp.zeros((STRIPE, DP), jnp.float32)
  acc_u, acc_i = _sc_segment_sums(xu_p, xi_p, idx, nrm, zeros)
  bsum = (W1_b + W2_b).reshape(1, D)
  return _tc_finish(acc_u, acc_i, x_user, x_item, W1_w, W2_w, bsum)
